# Initial kernel scaffold; baseline (speedup 1.0000x reference)
import jax
import jax.numpy as jnp
from jax.experimental import pallas as pl


def kernel(x, edge_index, W_gat, a_src, a_dst, b_gat, W1, b1, W2, b2):
    raise NotImplementedError("write your pallas kernel here")


# speedup vs baseline: 11.3367x; 11.3367x over previous
import jax
import jax.numpy as jnp
from jax import lax
from jax.experimental import pallas as pl
from jax.experimental.pallas import tpu as pltpu
from jax.experimental.pallas import tpu_sc as plsc

N = 10000
E = 160000
IN = 256
OUT = 256
HALF = 128

NC = 2
NS = 16
EPT = E // NS
CH = 80
NCHUNK = EPT // CH
NPT = 640
NPAD = NS * NPT
LAST_NPT = 400
ROWCH = 80

RB = 1000
GRID = N // RB


def _tc1_body(x_ref, wext_ref, h0_ref, h1_ref, sd_ref):
    he = jnp.dot(x_ref[...], wext_ref[...], preferred_element_type=jnp.float32)
    h0_ref[...] = he[:, :HALF]
    h1_ref[...] = he[:, HALF:OUT]
    sd_ref[...] = he[:, OUT:]


def _tc1(x, wext):
    return pl.pallas_call(
        _tc1_body,
        grid=(GRID,),
        in_specs=[
            pl.BlockSpec((RB, IN), lambda i: (i, 0)),
            pl.BlockSpec((IN, OUT + HALF), lambda i: (0, 0)),
        ],
        out_specs=[
            pl.BlockSpec((RB, HALF), lambda i: (i, 0)),
            pl.BlockSpec((RB, HALF), lambda i: (i, 0)),
            pl.BlockSpec((RB, HALF), lambda i: (i, 0)),
        ],
        out_shape=[
            jax.ShapeDtypeStruct((N, HALF), jnp.float32),
            jax.ShapeDtypeStruct((N, HALF), jnp.float32),
            jax.ShapeDtypeStruct((N, HALF), jnp.float32),
        ],
    )(x, wext)


def _sc_body(h0, h1, s_hbm, d_hbm, src_hbm, dst2_hbm,
             o0, o1,
             src_f, dst2, sval, dval, w_buf, rows_buf, zvec, den_slice,
             sh_s, sh_d, sh_den, sh_out, sem):
    c = lax.axis_index("c")
    t = lax.axis_index("s")

    pltpu.sync_copy(src_hbm.at[pl.ds(t * EPT, EPT)], src_f)
    pltpu.sync_copy(dst2_hbm.at[t], dst2)

    node_base = t * NPT
    nq = jnp.where(t == NS - 1, LAST_NPT // ROWCH, NPT // ROWCH)

    pltpu.sync_copy(s_hbm.at[pl.ds(node_base, NPT)],
                    sh_s.at[pl.ds(node_base, NPT)])
    pltpu.sync_copy(d_hbm.at[pl.ds(node_base, NPT)],
                    sh_d.at[pl.ds(node_base, NPT)])

    zeros16 = jnp.zeros((16,), jnp.float32)

    def _zero_zvec(i, carry):
        zvec[pl.ds(i * 16, 16)] = zeros16
        return carry

    lax.fori_loop(0, NPT // 16, _zero_zvec, 0)

    def _zero_rows(i, carry):
        for k in range(HALF // 16):
            rows_buf[i, pl.ds(k * 16, 16)] = zeros16
        return carry

    lax.fori_loop(0, CH, _zero_rows, 0)

    pltpu.sync_copy(zvec, sh_den.at[pl.ds(node_base, NPT)])

    def _zero_out(q, carry):
        pltpu.sync_copy(rows_buf,
                        sh_out.at[pl.ds(node_base + q * ROWCH, ROWCH)])
        return carry

    lax.fori_loop(0, nq, _zero_out, 0)

    plsc.subcore_barrier()

    def _process(h_hbm):
        def _chunk(j, carry):
            sidx = src_f.at[pl.ds(j * CH, CH)]
            didx = dst2.at[j]
            pltpu.sync_copy(sh_s.at[sidx], sval)
            pltpu.sync_copy(sh_d.at[didx], dval)
            for k in range(CH // 16):
                sl = pl.ds(k * 16, 16)
                e = sval[sl] + dval[sl]
                e = jnp.where(e > 0, e, e * 0.2)
                w_buf[sl] = jnp.exp(e)
            pltpu.sync_copy(w_buf, sh_den.at[didx], add=True)
            pltpu.async_copy(h_hbm.at[sidx], rows_buf, sem).wait()

            def _scale(i, carry2):
                wv = plsc.load_gather(w_buf, [jnp.full((16,), i, jnp.int32)])
                for k in range(HALF // 16):
                    sl = pl.ds(k * 16, 16)
                    rows_buf[i, sl] = rows_buf[i, sl] * wv
                return carry2

            lax.fori_loop(0, CH, _scale, 0)
            pltpu.sync_copy(rows_buf, sh_out.at[didx], add=True)
            return carry

        lax.fori_loop(0, NCHUNK, _chunk, 0)

    @pl.when(c == 0)
    def _():
        _process(h0)

    @pl.when(c == 1)
    def _():
        _process(h1)

    plsc.subcore_barrier()

    pltpu.sync_copy(sh_den.at[pl.ds(node_base, NPT)], den_slice)

    def _drain(o_hbm):
        def _chunk(q, carry):
            rowbase = node_base + q * ROWCH
            pltpu.sync_copy(sh_out.at[pl.ds(rowbase, ROWCH)], rows_buf)

            def _div(i, carry2):
                dv = plsc.load_gather(
                    den_slice, [jnp.full((16,), q * ROWCH + i, jnp.int32)])
                dv = jnp.maximum(dv, 1e-30)
                for k in range(HALF // 16):
                    sl = pl.ds(k * 16, 16)
                    rows_buf[i, sl] = rows_buf[i, sl] / dv
                return carry2

            lax.fori_loop(0, ROWCH, _div, 0)
            pltpu.sync_copy(rows_buf, o_hbm.at[pl.ds(rowbase, ROWCH)])
            return carry

        lax.fori_loop(0, nq, _chunk, 0)

    @pl.when(c == 0)
    def _():
        _drain(o0)

    @pl.when(c == 1)
    def _():
        _drain(o1)


def _sc_gat(h0, h1, s, d, src, dst2):
    mesh = plsc.VectorSubcoreMesh(core_axis_name="c", subcore_axis_name="s",
                                  num_cores=NC, num_subcores=NS)
    f = pl.kernel(
        _sc_body,
        out_type=[
            jax.ShapeDtypeStruct((N, HALF), jnp.float32),
            jax.ShapeDtypeStruct((N, HALF), jnp.float32),
        ],
        mesh=mesh,
        compiler_params=pltpu.CompilerParams(needs_layout_passes=False),
        scratch_types=[
            pltpu.VMEM((EPT,), jnp.int32),
            pltpu.VMEM((NCHUNK, CH), jnp.int32),
            pltpu.VMEM((CH,), jnp.float32),
            pltpu.VMEM((CH,), jnp.float32),
            pltpu.VMEM((CH,), jnp.float32),
            pltpu.VMEM((CH, HALF), jnp.float32),
            pltpu.VMEM((NPT,), jnp.float32),
            pltpu.VMEM((NPT,), jnp.float32),
            pltpu.VMEM_SHARED((NPAD,), jnp.float32),
            pltpu.VMEM_SHARED((NPAD,), jnp.float32),
            pltpu.VMEM_SHARED((NPAD,), jnp.float32),
            pltpu.VMEM_SHARED((N, HALF), jnp.float32),
            pltpu.SemaphoreType.DMA,
        ],
    )
    return f(h0, h1, s, d, src, dst2)


def _tc2_body(o0_ref, o1_ref, bg_ref, w1_ref, b1_ref, w2_ref, b2_ref, y_ref):
    g = jnp.concatenate([o0_ref[...], o1_ref[...]], axis=1) + bg_ref[...]
    g = jnp.where(g > 0, g, jnp.exp(g) - 1.0)
    z = jnp.dot(g, w1_ref[...], preferred_element_type=jnp.float32)
    z = jnp.maximum(z + b1_ref[...], 0.0)
    y = jnp.dot(z, w2_ref[...], preferred_element_type=jnp.float32)
    y_ref[...] = y + b2_ref[...]


def _tc2(o0, o1, bg, w1p, b1p, w2p, b2p):
    return pl.pallas_call(
        _tc2_body,
        grid=(GRID,),
        in_specs=[
            pl.BlockSpec((RB, HALF), lambda i: (i, 0)),
            pl.BlockSpec((RB, HALF), lambda i: (i, 0)),
            pl.BlockSpec((1, OUT), lambda i: (0, 0)),
            pl.BlockSpec((OUT, HALF), lambda i: (0, 0)),
            pl.BlockSpec((1, HALF), lambda i: (0, 0)),
            pl.BlockSpec((HALF, HALF), lambda i: (0, 0)),
            pl.BlockSpec((1, HALF), lambda i: (0, 0)),
        ],
        out_specs=pl.BlockSpec((RB, HALF), lambda i: (i, 0)),
        out_shape=jax.ShapeDtypeStruct((N, HALF), jnp.float32),
    )(o0, o1, bg, w1p, b1p, w2p, b2p)


@jax.jit
def kernel(x, edge_index, W_gat, a_src, a_dst, b_gat, W1, b1, W2, b2):
    src = edge_index[0]
    dst = edge_index[1]

    wext = jnp.concatenate(
        [W_gat, (W_gat @ a_src)[:, None], (W_gat @ a_dst)[:, None],
         jnp.zeros((IN, HALF - 2), jnp.float32)], axis=1)
    h1cols = W1.shape[1]
    w1p = jnp.pad(W1, ((0, 0), (0, HALF - h1cols)))
    b1p = jnp.pad(b1, (0, HALF - h1cols))[None, :]
    w2p = jnp.pad(W2, ((0, HALF - h1cols), (0, HALF - W2.shape[1])))
    b2p = jnp.pad(b2, (0, HALF - W2.shape[1]))[None, :]
    bg = b_gat[None, :]

    h0, h1, sd = _tc1(x, wext)
    s = jnp.pad(sd[:, 0], (0, NPAD - N))
    d = jnp.pad(sd[:, 1], (0, NPAD - N))
    dst2 = dst.reshape(NS, NCHUNK, CH)

    o0, o1 = _sc_gat(h0, h1, s, d, src, dst2)

    y = _tc2(o0, o1, bg, w1p, b1p, w2p, b2p)
    return y[:, :W2.shape[1]]


# speedup vs baseline: 16.4499x; 1.4510x over previous
import functools

import jax
import jax.numpy as jnp
from jax import lax
from jax.experimental import pallas as pl
from jax.experimental.pallas import tpu as pltpu
from jax.experimental.pallas import tpu_sc as plsc

N = 10000
E = 160000
IN = 256
OUT = 256
HALF = 128

NC = 2
NS = 16
EPT = E // NS
CH = 80
NCHUNK = EPT // CH
NPT = 640
NPAD = NS * NPT
LAST_NPT = 400
ROWCH = 80

RB = 1000
GRID = N // RB


def _tc1_body(x_ref, wext_ref, h0_ref, h1_ref, sd_ref):
    he = jnp.dot(x_ref[...], wext_ref[...], preferred_element_type=jnp.float32)
    h0_ref[...] = he[:, :HALF]
    h1_ref[...] = he[:, HALF:OUT]
    sd_ref[...] = he[:, OUT:]


def _tc1(x, wext):
    return pl.pallas_call(
        _tc1_body,
        grid=(GRID,),
        in_specs=[
            pl.BlockSpec((RB, IN), lambda i: (i, 0)),
            pl.BlockSpec((IN, OUT + HALF), lambda i: (0, 0)),
        ],
        out_specs=[
            pl.BlockSpec((RB, HALF), lambda i: (i, 0)),
            pl.BlockSpec((RB, HALF), lambda i: (i, 0)),
            pl.BlockSpec((RB, HALF), lambda i: (i, 0)),
        ],
        out_shape=[
            jax.ShapeDtypeStruct((N, HALF), jnp.float32),
            jax.ShapeDtypeStruct((N, HALF), jnp.float32),
            jax.ShapeDtypeStruct((N, HALF), jnp.float32),
        ],
    )(x, wext)


def _sc_body(h0, h1, s_hbm, d_hbm, src_hbm, dst2_hbm,
             o0, o1, den_hbm,
             src_f, dst2, sval, dval, w_buf, rows_a, rows_b, zvec,
             sh_s, sh_d, sh_den, sh_out, sem_a, sem_b):
    c = lax.axis_index("c")
    t = lax.axis_index("s")

    pltpu.sync_copy(src_hbm.at[pl.ds(t * EPT, EPT)], src_f)
    pltpu.sync_copy(dst2_hbm.at[t], dst2)

    node_base = t * NPT
    nq = jnp.where(t == NS - 1, LAST_NPT // ROWCH, NPT // ROWCH)

    pltpu.sync_copy(s_hbm.at[pl.ds(node_base, NPT)],
                    sh_s.at[pl.ds(node_base, NPT)])
    pltpu.sync_copy(d_hbm.at[pl.ds(node_base, NPT)],
                    sh_d.at[pl.ds(node_base, NPT)])

    zeros16 = jnp.zeros((16,), jnp.float32)

    def _zero_zvec(i, carry):
        zvec[pl.ds(i * 16, 16)] = zeros16
        return carry

    lax.fori_loop(0, NPT // 16, _zero_zvec, 0)

    def _zero_rows(i, carry):
        for k in range(HALF // 16):
            rows_a[i, pl.ds(k * 16, 16)] = zeros16
        return carry

    lax.fori_loop(0, CH, _zero_rows, 0)

    pltpu.sync_copy(zvec, sh_den.at[pl.ds(node_base, NPT)])

    def _zero_out(q, carry):
        pltpu.sync_copy(rows_a,
                        sh_out.at[pl.ds(node_base + q * ROWCH, ROWCH)])
        return carry

    lax.fori_loop(0, nq, _zero_out, 0)

    plsc.subcore_barrier()

    def _process(h_hbm):
        pltpu.async_copy(h_hbm.at[src_f.at[pl.ds(0, CH)]], rows_a, sem_a)

        def _outer(p, carry):
            for b in range(2):
                j = p * 2 + b
                buf = rows_a if b == 0 else rows_b
                sem = sem_a if b == 0 else sem_b
                nbuf = rows_b if b == 0 else rows_a
                nsem = sem_b if b == 0 else sem_a

                @pl.when(j < NCHUNK)
                def _():
                    sidx = src_f.at[pl.ds(j * CH, CH)]
                    didx = dst2.at[j]
                    @pl.when(j + 1 < NCHUNK)
                    def _():
                        nidx = src_f.at[pl.ds((j + 1) * CH, CH)]
                        pltpu.async_copy(h_hbm.at[nidx], nbuf, nsem)

                    pltpu.sync_copy(sh_s.at[sidx], sval)
                    pltpu.sync_copy(sh_d.at[didx], dval)
                    for k in range(CH // 16):
                        sl = pl.ds(k * 16, 16)
                        e = sval[sl] + dval[sl]
                        e = jnp.where(e > 0, e, e * 0.2)
                        w_buf[sl] = jnp.exp(e)
                    pltpu.sync_copy(w_buf, sh_den.at[didx], add=True)

                    pltpu.make_async_copy(h_hbm.at[sidx], buf, sem).wait()

                    def _scale(i, carry2):
                        wv = plsc.load_gather(
                            w_buf, [jnp.full((16,), i, jnp.int32)])
                        for k in range(HALF // 16):
                            sl = pl.ds(k * 16, 16)
                            buf[i, sl] = buf[i, sl] * wv
                        return carry2

                    lax.fori_loop(0, CH, _scale, 0)

                    pltpu.sync_copy(buf, sh_out.at[didx], add=True)
            return carry

        lax.fori_loop(0, (NCHUNK + 1) // 2, _outer, 0)

    @pl.when(c == 0)
    def _():
        _process(h0)

    @pl.when(c == 1)
    def _():
        _process(h1)

    plsc.subcore_barrier()

    @pl.when(c == 0)
    def _():
        pltpu.sync_copy(sh_den.at[pl.ds(node_base, NPT)],
                        den_hbm.at[pl.ds(node_base, NPT)])

    def _drain(o_hbm):
        @pl.when(t < NS - 1)
        def _():
            pltpu.sync_copy(sh_out.at[pl.ds(node_base, NPT)],
                            o_hbm.at[pl.ds(node_base, NPT)])

        @pl.when(t == NS - 1)
        def _():
            pltpu.sync_copy(sh_out.at[pl.ds((NS - 1) * NPT, LAST_NPT)],
                            o_hbm.at[pl.ds((NS - 1) * NPT, LAST_NPT)])

    @pl.when(c == 0)
    def _():
        _drain(o0)

    @pl.when(c == 1)
    def _():
        _drain(o1)


def _sc_gat(h0, h1, s, d, src, dst2):
    mesh = plsc.VectorSubcoreMesh(core_axis_name="c", subcore_axis_name="s",
                                  num_cores=NC, num_subcores=NS)
    f = pl.kernel(
        _sc_body,
        out_type=[
            jax.ShapeDtypeStruct((N, HALF), jnp.float32),
            jax.ShapeDtypeStruct((N, HALF), jnp.float32),
            jax.ShapeDtypeStruct((NPAD,), jnp.float32),
        ],
        mesh=mesh,
        compiler_params=pltpu.CompilerParams(needs_layout_passes=False),
        scratch_types=[
            pltpu.VMEM((EPT,), jnp.int32),
            pltpu.VMEM((NCHUNK, CH), jnp.int32),
            pltpu.VMEM((CH,), jnp.float32),
            pltpu.VMEM((CH,), jnp.float32),
            pltpu.VMEM((CH,), jnp.float32),
            pltpu.VMEM((CH, HALF), jnp.float32),
            pltpu.VMEM((CH, HALF), jnp.float32),
            pltpu.VMEM((NPT,), jnp.float32),
            pltpu.VMEM_SHARED((NPAD,), jnp.float32),
            pltpu.VMEM_SHARED((NPAD,), jnp.float32),
            pltpu.VMEM_SHARED((NPAD,), jnp.float32),
            pltpu.VMEM_SHARED((N, HALF), jnp.float32),
            pltpu.SemaphoreType.DMA,
            pltpu.SemaphoreType.DMA,
        ],
    )
    return f(h0, h1, s, d, src, dst2)


def _tc2_body(o0_ref, o1_ref, den_ref, bg_ref, w1_ref, b1_ref, w2_ref,
              b2_ref, y_ref):
    recip = 1.0 / jnp.maximum(den_ref[...], 1e-30)
    g = jnp.concatenate([o0_ref[...], o1_ref[...]], axis=1)
    g = g * recip + bg_ref[...]
    g = jnp.where(g > 0, g, jnp.exp(g) - 1.0)
    z = jnp.dot(g, w1_ref[...], preferred_element_type=jnp.float32)
    z = jnp.maximum(z + b1_ref[...], 0.0)
    y = jnp.dot(z, w2_ref[...], preferred_element_type=jnp.float32)
    y_ref[...] = y + b2_ref[...]


def _tc2(o0, o1, den, bg, w1p, b1p, w2p, b2p):
    return pl.pallas_call(
        _tc2_body,
        grid=(GRID,),
        in_specs=[
            pl.BlockSpec((RB, HALF), lambda i: (i, 0)),
            pl.BlockSpec((RB, HALF), lambda i: (i, 0)),
            pl.BlockSpec((RB, 1), lambda i: (i, 0)),
            pl.BlockSpec((1, OUT), lambda i: (0, 0)),
            pl.BlockSpec((OUT, HALF), lambda i: (0, 0)),
            pl.BlockSpec((1, HALF), lambda i: (0, 0)),
            pl.BlockSpec((HALF, HALF), lambda i: (0, 0)),
            pl.BlockSpec((1, HALF), lambda i: (0, 0)),
        ],
        out_specs=pl.BlockSpec((RB, HALF), lambda i: (i, 0)),
        out_shape=jax.ShapeDtypeStruct((N, HALF), jnp.float32),
    )(o0, o1, den, bg, w1p, b1p, w2p, b2p)


@jax.jit
def kernel(x, edge_index, W_gat, a_src, a_dst, b_gat, W1, b1, W2, b2):
    src = edge_index[0]
    dst = edge_index[1]

    wext = jnp.concatenate(
        [W_gat, (W_gat @ a_src)[:, None], (W_gat @ a_dst)[:, None],
         jnp.zeros((IN, HALF - 2), jnp.float32)], axis=1)
    h1cols = W1.shape[1]
    w1p = jnp.pad(W1, ((0, 0), (0, HALF - h1cols)))
    b1p = jnp.pad(b1, (0, HALF - h1cols))[None, :]
    w2p = jnp.pad(W2, ((0, HALF - h1cols), (0, HALF - W2.shape[1])))
    b2p = jnp.pad(b2, (0, HALF - W2.shape[1]))[None, :]
    bg = b_gat[None, :]

    h0, h1, sd = _tc1(x, wext)
    s = jnp.pad(sd[:, 0], (0, NPAD - N))
    d = jnp.pad(sd[:, 1], (0, NPAD - N))
    dst2 = dst.reshape(NS, NCHUNK, CH)

    o0, o1, den = _sc_gat(h0, h1, s, d, src, dst2)

    y = _tc2(o0, o1, den[:N, None], bg, w1p, b1p, w2p, b2p)
    return y[:, :W2.shape[1]]


# speedup vs baseline: 16.9835x; 1.0324x over previous
import functools

import jax
import jax.numpy as jnp
from jax import lax
from jax.experimental import pallas as pl
from jax.experimental.pallas import tpu as pltpu
from jax.experimental.pallas import tpu_sc as plsc

N = 10000
E = 160000
IN = 256
OUT = 256
HALF = 128

NC = 2
NS = 16
EPT = E // NS
CH = 80
NCHUNK = EPT // CH
NPT = 640
NPAD = NS * NPT
LAST_NPT = 400
ROWCH = 80

RB = 1000
GRID = N // RB


def _tc1_body(x_ref, wext_ref, h0_ref, h1_ref, sd_ref):
    he = jnp.dot(x_ref[...], wext_ref[...], preferred_element_type=jnp.float32)
    h0_ref[...] = he[:, :HALF]
    h1_ref[...] = he[:, HALF:OUT]
    sd_ref[...] = he[:, OUT:]


def _tc1(x, wext):
    return pl.pallas_call(
        _tc1_body,
        grid=(GRID,),
        in_specs=[
            pl.BlockSpec((RB, IN), lambda i: (i, 0)),
            pl.BlockSpec((IN, OUT + HALF), lambda i: (0, 0)),
        ],
        out_specs=[
            pl.BlockSpec((RB, HALF), lambda i: (i, 0)),
            pl.BlockSpec((RB, HALF), lambda i: (i, 0)),
            pl.BlockSpec((RB, HALF), lambda i: (i, 0)),
        ],
        out_shape=[
            jax.ShapeDtypeStruct((N, HALF), jnp.float32),
            jax.ShapeDtypeStruct((N, HALF), jnp.float32),
            jax.ShapeDtypeStruct((N, HALF), jnp.float32),
        ],
    )(x, wext)


def _sc_body(h0, h1, s_hbm, d_hbm, src_hbm, dst2_hbm,
             o0, o1, den_hbm,
             src_f, dst2, sval, dval, w_buf, rows_a, rows_b, zvec,
             sh_s, sh_d, sh_den, sh_out, sem_a, sem_b):
    c = lax.axis_index("c")
    t = lax.axis_index("s")

    pltpu.sync_copy(src_hbm.at[pl.ds(t * EPT, EPT)], src_f)
    pltpu.sync_copy(dst2_hbm.at[t], dst2)

    node_base = t * NPT
    nq = jnp.where(t == NS - 1, LAST_NPT // ROWCH, NPT // ROWCH)

    pltpu.sync_copy(s_hbm.at[pl.ds(node_base, NPT)],
                    sh_s.at[pl.ds(node_base, NPT)])
    pltpu.sync_copy(d_hbm.at[pl.ds(node_base, NPT)],
                    sh_d.at[pl.ds(node_base, NPT)])

    zeros16 = jnp.zeros((16,), jnp.float32)

    def _zero_zvec(i, carry):
        zvec[pl.ds(i * 16, 16)] = zeros16
        return carry

    lax.fori_loop(0, NPT // 16, _zero_zvec, 0)

    def _zero_rows(i, carry):
        for k in range(HALF // 16):
            rows_a[i, pl.ds(k * 16, 16)] = zeros16
        return carry

    lax.fori_loop(0, CH, _zero_rows, 0)

    pltpu.sync_copy(zvec, sh_den.at[pl.ds(node_base, NPT)])

    def _zero_out(q, carry):
        pltpu.sync_copy(rows_a,
                        sh_out.at[pl.ds(node_base + q * ROWCH, ROWCH)])
        return carry

    lax.fori_loop(0, nq, _zero_out, 0)

    plsc.subcore_barrier()

    def _process(h_hbm):
        pltpu.async_copy(h_hbm.at[src_f.at[pl.ds(0, CH)]], rows_a, sem_a)

        def _outer(p, carry):
            for b in range(2):
                j = p * 2 + b
                buf = rows_a if b == 0 else rows_b
                sem = sem_a if b == 0 else sem_b
                nbuf = rows_b if b == 0 else rows_a
                nsem = sem_b if b == 0 else sem_a

                @pl.when(j < NCHUNK)
                def _():
                    sidx = src_f.at[pl.ds(j * CH, CH)]
                    didx = dst2.at[j]
                    @pl.when(j + 1 < NCHUNK)
                    def _():
                        nidx = src_f.at[pl.ds((j + 1) * CH, CH)]
                        pltpu.async_copy(h_hbm.at[nidx], nbuf, nsem)

                    pltpu.sync_copy(sh_s.at[sidx], sval)
                    pltpu.sync_copy(sh_d.at[didx], dval)
                    for k in range(CH // 16):
                        sl = pl.ds(k * 16, 16)
                        e = sval[sl] + dval[sl]
                        e = jnp.where(e > 0, e, e * 0.2)
                        w_buf[sl] = jnp.exp(e)
                    pltpu.sync_copy(w_buf, sh_den.at[didx], add=True)

                    pltpu.make_async_copy(h_hbm.at[sidx], buf, sem).wait()

                    def _scale(i4, carry2):
                        for r in range(4):
                            i = i4 * 4 + r
                            wv = plsc.load_gather(
                                w_buf, [jnp.full((16,), i, jnp.int32)])
                            for k in range(HALF // 16):
                                sl = pl.ds(k * 16, 16)
                                buf[i, sl] = buf[i, sl] * wv
                        return carry2

                    lax.fori_loop(0, CH // 4, _scale, 0)

                    pltpu.sync_copy(buf, sh_out.at[didx], add=True)
            return carry

        lax.fori_loop(0, (NCHUNK + 1) // 2, _outer, 0)

    @pl.when(c == 0)
    def _():
        _process(h0)

    @pl.when(c == 1)
    def _():
        _process(h1)

    plsc.subcore_barrier()

    @pl.when(c == 0)
    def _():
        pltpu.sync_copy(sh_den.at[pl.ds(node_base, NPT)],
                        den_hbm.at[pl.ds(node_base, NPT)])

    def _drain(o_hbm):
        @pl.when(t < NS - 1)
        def _():
            pltpu.sync_copy(sh_out.at[pl.ds(node_base, NPT)],
                            o_hbm.at[pl.ds(node_base, NPT)])

        @pl.when(t == NS - 1)
        def _():
            pltpu.sync_copy(sh_out.at[pl.ds((NS - 1) * NPT, LAST_NPT)],
                            o_hbm.at[pl.ds((NS - 1) * NPT, LAST_NPT)])

    @pl.when(c == 0)
    def _():
        _drain(o0)

    @pl.when(c == 1)
    def _():
        _drain(o1)


def _sc_gat(h0, h1, s, d, src, dst2):
    mesh = plsc.VectorSubcoreMesh(core_axis_name="c", subcore_axis_name="s",
                                  num_cores=NC, num_subcores=NS)
    f = pl.kernel(
        _sc_body,
        out_type=[
            jax.ShapeDtypeStruct((N, HALF), jnp.float32),
            jax.ShapeDtypeStruct((N, HALF), jnp.float32),
            jax.ShapeDtypeStruct((NPAD,), jnp.float32),
        ],
        mesh=mesh,
        compiler_params=pltpu.CompilerParams(needs_layout_passes=False),
        scratch_types=[
            pltpu.VMEM((EPT,), jnp.int32),
            pltpu.VMEM((NCHUNK, CH), jnp.int32),
            pltpu.VMEM((CH,), jnp.float32),
            pltpu.VMEM((CH,), jnp.float32),
            pltpu.VMEM((CH,), jnp.float32),
            pltpu.VMEM((CH, HALF), jnp.float32),
            pltpu.VMEM((CH, HALF), jnp.float32),
            pltpu.VMEM((NPT,), jnp.float32),
            pltpu.VMEM_SHARED((NPAD,), jnp.float32),
            pltpu.VMEM_SHARED((NPAD,), jnp.float32),
            pltpu.VMEM_SHARED((NPAD,), jnp.float32),
            pltpu.VMEM_SHARED((N, HALF), jnp.float32),
            pltpu.SemaphoreType.DMA,
            pltpu.SemaphoreType.DMA,
        ],
    )
    return f(h0, h1, s, d, src, dst2)


def _tc2_body(o0_ref, o1_ref, den_ref, bg_ref, w1_ref, b1_ref, w2_ref,
              b2_ref, y_ref):
    recip = 1.0 / jnp.maximum(den_ref[...], 1e-30)
    g = jnp.concatenate([o0_ref[...], o1_ref[...]], axis=1)
    g = g * recip + bg_ref[...]
    g = jnp.where(g > 0, g, jnp.exp(g) - 1.0)
    z = jnp.dot(g, w1_ref[...], preferred_element_type=jnp.float32)
    z = jnp.maximum(z + b1_ref[...], 0.0)
    y = jnp.dot(z, w2_ref[...], preferred_element_type=jnp.float32)
    y_ref[...] = y[:, :8] + b2_ref[...]


def _tc2(o0, o1, den, bg, w1p, b1p, w2p, b2p):
    return pl.pallas_call(
        _tc2_body,
        grid=(GRID,),
        in_specs=[
            pl.BlockSpec((RB, HALF), lambda i: (i, 0)),
            pl.BlockSpec((RB, HALF), lambda i: (i, 0)),
            pl.BlockSpec((RB, 1), lambda i: (i, 0)),
            pl.BlockSpec((1, OUT), lambda i: (0, 0)),
            pl.BlockSpec((OUT, HALF), lambda i: (0, 0)),
            pl.BlockSpec((1, HALF), lambda i: (0, 0)),
            pl.BlockSpec((HALF, HALF), lambda i: (0, 0)),
            pl.BlockSpec((1, 8), lambda i: (0, 0)),
        ],
        out_specs=pl.BlockSpec((RB, 8), lambda i: (i, 0)),
        out_shape=jax.ShapeDtypeStruct((N, 8), jnp.float32),
    )(o0, o1, den, bg, w1p, b1p, w2p, b2p)


@jax.jit
def kernel(x, edge_index, W_gat, a_src, a_dst, b_gat, W1, b1, W2, b2):
    src = edge_index[0]
    dst = edge_index[1]

    wext = jnp.concatenate(
        [W_gat, (W_gat @ a_src)[:, None], (W_gat @ a_dst)[:, None],
         jnp.zeros((IN, HALF - 2), jnp.float32)], axis=1)
    h1cols = W1.shape[1]
    w1p = jnp.pad(W1, ((0, 0), (0, HALF - h1cols)))
    b1p = jnp.pad(b1, (0, HALF - h1cols))[None, :]
    w2p = jnp.pad(W2, ((0, HALF - h1cols), (0, HALF - W2.shape[1])))
    b2p = jnp.pad(b2, (0, 8 - W2.shape[1]))[None, :]
    bg = b_gat[None, :]

    h0, h1, sd = _tc1(x, wext)
    s = jnp.pad(sd[:, 0], (0, NPAD - N))
    d = jnp.pad(sd[:, 1], (0, NPAD - N))
    dst2 = dst.reshape(NS, NCHUNK, CH)

    o0, o1, den = _sc_gat(h0, h1, s, d, src, dst2)

    y = _tc2(o0, o1, den[:N, None], bg, w1p, b1p, w2p, b2p)
    return y[:, :W2.shape[1]]


# speedup vs baseline: 17.0026x; 1.0011x over previous
import functools

import jax
import jax.numpy as jnp
from jax import lax
from jax.experimental import pallas as pl
from jax.experimental.pallas import tpu as pltpu
from jax.experimental.pallas import tpu_sc as plsc

N = 10000
E = 160000
IN = 256
OUT = 256
HALF = 128

NC = 2
NS = 16
EPT = E // NS
CH = 80
NCHUNK = EPT // CH
NPT = 640
NPAD = NS * NPT
LAST_NPT = 400
ROWCH = 80

RB = 1000
GRID = N // RB


def _tc1_body(x_ref, wext_ref, h0_ref, h1_ref, sd_ref):
    he = jnp.dot(x_ref[...], wext_ref[...], preferred_element_type=jnp.float32)
    h0_ref[...] = he[:, :HALF]
    h1_ref[...] = he[:, HALF:OUT]
    sd_ref[...] = he[:, OUT:]


def _tc1(x, wext):
    return pl.pallas_call(
        _tc1_body,
        grid=(GRID,),
        in_specs=[
            pl.BlockSpec((RB, IN), lambda i: (i, 0)),
            pl.BlockSpec((IN, OUT + HALF), lambda i: (0, 0)),
        ],
        out_specs=[
            pl.BlockSpec((RB, HALF), lambda i: (i, 0)),
            pl.BlockSpec((RB, HALF), lambda i: (i, 0)),
            pl.BlockSpec((RB, HALF), lambda i: (i, 0)),
        ],
        out_shape=[
            jax.ShapeDtypeStruct((N, HALF), jnp.float32),
            jax.ShapeDtypeStruct((N, HALF), jnp.float32),
            jax.ShapeDtypeStruct((N, HALF), jnp.float32),
        ],
    )(x, wext)


def _sc_body(h0, h1, s_hbm, d_hbm, src_hbm, dst2_hbm,
             o0, o1, den_hbm,
             src_f, dst2, sval, dval, w_buf, rows_a, rows_b, zvec,
             sh_s, sh_d, sh_den, sh_out, sem_a, sem_b, ssem_a, ssem_b):
    c = lax.axis_index("c")
    t = lax.axis_index("s")

    pltpu.sync_copy(src_hbm.at[pl.ds(t * EPT, EPT)], src_f)
    pltpu.sync_copy(dst2_hbm.at[t], dst2)

    node_base = t * NPT
    nq = jnp.where(t == NS - 1, LAST_NPT // ROWCH, NPT // ROWCH)

    pltpu.sync_copy(s_hbm.at[pl.ds(node_base, NPT)],
                    sh_s.at[pl.ds(node_base, NPT)])
    pltpu.sync_copy(d_hbm.at[pl.ds(node_base, NPT)],
                    sh_d.at[pl.ds(node_base, NPT)])

    zeros16 = jnp.zeros((16,), jnp.float32)

    def _zero_zvec(i, carry):
        zvec[pl.ds(i * 16, 16)] = zeros16
        return carry

    lax.fori_loop(0, NPT // 16, _zero_zvec, 0)

    def _zero_rows(i, carry):
        for k in range(HALF // 16):
            rows_a[i, pl.ds(k * 16, 16)] = zeros16
        return carry

    lax.fori_loop(0, CH, _zero_rows, 0)

    pltpu.sync_copy(zvec, sh_den.at[pl.ds(node_base, NPT)])

    def _zero_out(q, carry):
        pltpu.sync_copy(rows_a,
                        sh_out.at[pl.ds(node_base + q * ROWCH, ROWCH)])
        return carry

    lax.fori_loop(0, nq, _zero_out, 0)

    plsc.subcore_barrier()

    def _process(h_hbm):
        pltpu.async_copy(h_hbm.at[src_f.at[pl.ds(0, CH)]], rows_a, sem_a)

        def _outer(p, carry):
            for b in range(2):
                j = p * 2 + b
                buf = rows_a if b == 0 else rows_b
                sem = sem_a if b == 0 else sem_b
                ssem = ssem_a if b == 0 else ssem_b
                nbuf = rows_b if b == 0 else rows_a
                nsem = sem_b if b == 0 else sem_a
                psem = ssem_b if b == 0 else ssem_a

                @pl.when(j < NCHUNK)
                def _():
                    sidx = src_f.at[pl.ds(j * CH, CH)]
                    didx = dst2.at[j]

                    @pl.when(j >= 1)
                    def _():
                        pidx = dst2.at[j - 1]
                        pltpu.make_async_copy(nbuf, sh_out.at[pidx],
                                              psem).wait()

                    @pl.when(j + 1 < NCHUNK)
                    def _():
                        nidx = src_f.at[pl.ds((j + 1) * CH, CH)]
                        pltpu.async_copy(h_hbm.at[nidx], nbuf, nsem)

                    pltpu.sync_copy(sh_s.at[sidx], sval)
                    pltpu.sync_copy(sh_d.at[didx], dval)
                    for k in range(CH // 16):
                        sl = pl.ds(k * 16, 16)
                        e = sval[sl] + dval[sl]
                        e = jnp.where(e > 0, e, e * 0.2)
                        w_buf[sl] = jnp.exp(e)
                    pltpu.sync_copy(w_buf, sh_den.at[didx], add=True)

                    pltpu.make_async_copy(h_hbm.at[sidx], buf, sem).wait()

                    def _scale(i4, carry2):
                        for r in range(4):
                            i = i4 * 4 + r
                            wv = plsc.load_gather(
                                w_buf, [jnp.full((16,), i, jnp.int32)])
                            for k in range(HALF // 16):
                                sl = pl.ds(k * 16, 16)
                                buf[i, sl] = buf[i, sl] * wv
                        return carry2

                    lax.fori_loop(0, CH // 4, _scale, 0)

                    pltpu.async_copy(buf, sh_out.at[didx], ssem, add=True)
            return carry

        lax.fori_loop(0, (NCHUNK + 1) // 2, _outer, 0)
        pltpu.make_async_copy(rows_a, sh_out.at[dst2.at[NCHUNK - 1]],
                              ssem_a).wait()

    @pl.when(c == 0)
    def _():
        _process(h0)

    @pl.when(c == 1)
    def _():
        _process(h1)

    plsc.subcore_barrier()

    @pl.when(c == 0)
    def _():
        pltpu.sync_copy(sh_den.at[pl.ds(node_base, NPT)],
                        den_hbm.at[pl.ds(node_base, NPT)])

    def _drain(o_hbm):
        @pl.when(t < NS - 1)
        def _():
            pltpu.sync_copy(sh_out.at[pl.ds(node_base, NPT)],
                            o_hbm.at[pl.ds(node_base, NPT)])

        @pl.when(t == NS - 1)
        def _():
            pltpu.sync_copy(sh_out.at[pl.ds((NS - 1) * NPT, LAST_NPT)],
                            o_hbm.at[pl.ds((NS - 1) * NPT, LAST_NPT)])

    @pl.when(c == 0)
    def _():
        _drain(o0)

    @pl.when(c == 1)
    def _():
        _drain(o1)


def _sc_gat(h0, h1, s, d, src, dst2):
    mesh = plsc.VectorSubcoreMesh(core_axis_name="c", subcore_axis_name="s",
                                  num_cores=NC, num_subcores=NS)
    f = pl.kernel(
        _sc_body,
        out_type=[
            jax.ShapeDtypeStruct((N, HALF), jnp.float32),
            jax.ShapeDtypeStruct((N, HALF), jnp.float32),
            jax.ShapeDtypeStruct((NPAD,), jnp.float32),
        ],
        mesh=mesh,
        compiler_params=pltpu.CompilerParams(needs_layout_passes=False),
        scratch_types=[
            pltpu.VMEM((EPT,), jnp.int32),
            pltpu.VMEM((NCHUNK, CH), jnp.int32),
            pltpu.VMEM((CH,), jnp.float32),
            pltpu.VMEM((CH,), jnp.float32),
            pltpu.VMEM((CH,), jnp.float32),
            pltpu.VMEM((CH, HALF), jnp.float32),
            pltpu.VMEM((CH, HALF), jnp.float32),
            pltpu.VMEM((NPT,), jnp.float32),
            pltpu.VMEM_SHARED((NPAD,), jnp.float32),
            pltpu.VMEM_SHARED((NPAD,), jnp.float32),
            pltpu.VMEM_SHARED((NPAD,), jnp.float32),
            pltpu.VMEM_SHARED((N, HALF), jnp.float32),
            pltpu.SemaphoreType.DMA,
            pltpu.SemaphoreType.DMA,
            pltpu.SemaphoreType.DMA,
            pltpu.SemaphoreType.DMA,
        ],
    )
    return f(h0, h1, s, d, src, dst2)


def _tc2_body(o0_ref, o1_ref, den_ref, bg_ref, w1_ref, b1_ref, w2_ref,
              b2_ref, y_ref):
    recip = 1.0 / jnp.maximum(den_ref[...], 1e-30)
    g = jnp.concatenate([o0_ref[...], o1_ref[...]], axis=1)
    g = g * recip + bg_ref[...]
    g = jnp.where(g > 0, g, jnp.exp(g) - 1.0)
    z = jnp.dot(g, w1_ref[...], preferred_element_type=jnp.float32)
    z = jnp.maximum(z + b1_ref[...], 0.0)
    y = jnp.dot(z, w2_ref[...], preferred_element_type=jnp.float32)
    y_ref[...] = y[:, :8] + b2_ref[...]


def _tc2(o0, o1, den, bg, w1p, b1p, w2p, b2p):
    return pl.pallas_call(
        _tc2_body,
        grid=(GRID,),
        in_specs=[
            pl.BlockSpec((RB, HALF), lambda i: (i, 0)),
            pl.BlockSpec((RB, HALF), lambda i: (i, 0)),
            pl.BlockSpec((RB, 1), lambda i: (i, 0)),
            pl.BlockSpec((1, OUT), lambda i: (0, 0)),
            pl.BlockSpec((OUT, HALF), lambda i: (0, 0)),
            pl.BlockSpec((1, HALF), lambda i: (0, 0)),
            pl.BlockSpec((HALF, HALF), lambda i: (0, 0)),
            pl.BlockSpec((1, 8), lambda i: (0, 0)),
        ],
        out_specs=pl.BlockSpec((RB, 8), lambda i: (i, 0)),
        out_shape=jax.ShapeDtypeStruct((N, 8), jnp.float32),
    )(o0, o1, den, bg, w1p, b1p, w2p, b2p)


@jax.jit
def kernel(x, edge_index, W_gat, a_src, a_dst, b_gat, W1, b1, W2, b2):
    src = edge_index[0]
    dst = edge_index[1]

    wext = jnp.concatenate(
        [W_gat, (W_gat @ a_src)[:, None], (W_gat @ a_dst)[:, None],
         jnp.zeros((IN, HALF - 2), jnp.float32)], axis=1)
    h1cols = W1.shape[1]
    w1p = jnp.pad(W1, ((0, 0), (0, HALF - h1cols)))
    b1p = jnp.pad(b1, (0, HALF - h1cols))[None, :]
    w2p = jnp.pad(W2, ((0, HALF - h1cols), (0, HALF - W2.shape[1])))
    b2p = jnp.pad(b2, (0, 8 - W2.shape[1]))[None, :]
    bg = b_gat[None, :]

    h0, h1, sd = _tc1(x, wext)
    s = jnp.pad(sd[:, 0], (0, NPAD - N))
    d = jnp.pad(sd[:, 1], (0, NPAD - N))
    dst2 = dst.reshape(NS, NCHUNK, CH)

    o0, o1, den = _sc_gat(h0, h1, s, d, src, dst2)

    y = _tc2(o0, o1, den[:N, None], bg, w1p, b1p, w2p, b2p)
    return y[:, :W2.shape[1]]


# speedup vs baseline: 19.8098x; 1.1651x over previous
import functools

import jax
import jax.numpy as jnp
from jax import lax
from jax.experimental import pallas as pl
from jax.experimental.pallas import tpu as pltpu
from jax.experimental.pallas import tpu_sc as plsc

N = 10000
E = 160000
IN = 256
OUT = 256
HALF = 128

NC = 2
NS = 16
EPT = E // NS
CH = 80
NCHUNK = EPT // CH
NPT = 640
NPAD = NS * NPT
LAST_NPT = 400
ROWCH = 80

RB = 1000
GRID = N // RB


def _tc1_body(x_ref, wext_ref, h0_ref, h1_ref, sd_ref):
    he = jnp.dot(x_ref[...], wext_ref[...], preferred_element_type=jnp.float32)
    h0_ref[...] = he[:, :HALF]
    h1_ref[...] = he[:, HALF:OUT]
    sd_ref[...] = he[:, OUT:]


def _tc1(x, wext):
    return pl.pallas_call(
        _tc1_body,
        grid=(GRID,),
        in_specs=[
            pl.BlockSpec((RB, IN), lambda i: (i, 0)),
            pl.BlockSpec((IN, OUT + HALF), lambda i: (0, 0)),
        ],
        out_specs=[
            pl.BlockSpec((RB, HALF), lambda i: (i, 0)),
            pl.BlockSpec((RB, HALF), lambda i: (i, 0)),
            pl.BlockSpec((RB, HALF), lambda i: (i, 0)),
        ],
        out_shape=[
            jax.ShapeDtypeStruct((N, HALF), jnp.float32),
            jax.ShapeDtypeStruct((N, HALF), jnp.float32),
            jax.ShapeDtypeStruct((N, HALF), jnp.float32),
        ],
    )(x, wext)


def _sc_body(h0, h1, s_hbm, d_hbm, src_hbm, dst2_hbm,
             o0, o1, den_hbm,
             src_f, dst2, sval_a, sval_b, dval_a, dval_b, w_a, w_b,
             rows_a, rows_b, zvec,
             sh_s, sh_d, sh_den, sh_out,
             sem_a, sem_b, ssem_a, ssem_b, scsem_a, scsem_b,
             dnsem_a, dnsem_b):
    c = lax.axis_index("c")
    t = lax.axis_index("s")

    pltpu.sync_copy(src_hbm.at[pl.ds(t * EPT, EPT)], src_f)
    pltpu.sync_copy(dst2_hbm.at[t], dst2)

    node_base = t * NPT
    nq = jnp.where(t == NS - 1, LAST_NPT // ROWCH, NPT // ROWCH)

    pltpu.sync_copy(s_hbm.at[pl.ds(node_base, NPT)],
                    sh_s.at[pl.ds(node_base, NPT)])
    pltpu.sync_copy(d_hbm.at[pl.ds(node_base, NPT)],
                    sh_d.at[pl.ds(node_base, NPT)])

    zeros16 = jnp.zeros((16,), jnp.float32)

    def _zero_zvec(i, carry):
        zvec[pl.ds(i * 16, 16)] = zeros16
        return carry

    lax.fori_loop(0, NPT // 16, _zero_zvec, 0)

    def _zero_rows(i, carry):
        for k in range(HALF // 16):
            rows_a[i, pl.ds(k * 16, 16)] = zeros16
        return carry

    lax.fori_loop(0, CH, _zero_rows, 0)

    pltpu.sync_copy(zvec, sh_den.at[pl.ds(node_base, NPT)])

    def _zero_out(q, carry):
        pltpu.sync_copy(rows_a,
                        sh_out.at[pl.ds(node_base + q * ROWCH, ROWCH)])
        return carry

    lax.fori_loop(0, nq, _zero_out, 0)

    plsc.subcore_barrier()

    def _process(h_hbm):
        pltpu.async_copy(h_hbm.at[src_f.at[pl.ds(0, CH)]], rows_a, sem_a)
        pltpu.async_copy(sh_s.at[src_f.at[pl.ds(0, CH)]], sval_a, scsem_a)
        pltpu.async_copy(sh_d.at[dst2.at[0]], dval_a, scsem_a)

        def _outer(p, carry):
            for b in range(2):
                j = p * 2 + b
                buf, sem, ssem = ((rows_a, sem_a, ssem_a) if b == 0
                                  else (rows_b, sem_b, ssem_b))
                sv, dv, wb, scsem, dnsem = (
                    (sval_a, dval_a, w_a, scsem_a, dnsem_a) if b == 0
                    else (sval_b, dval_b, w_b, scsem_b, dnsem_b))
                nbuf, nsem = (rows_b, sem_b) if b == 0 else (rows_a, sem_a)
                nsv, ndv, nscsem = ((sval_b, dval_b, scsem_b) if b == 0
                                    else (sval_a, dval_a, scsem_a))
                psem = ssem_b if b == 0 else ssem_a

                @pl.when(j < NCHUNK)
                def _():
                    sidx = src_f.at[pl.ds(j * CH, CH)]
                    didx = dst2.at[j]

                    @pl.when(j >= 1)
                    def _():
                        pidx = dst2.at[j - 1]
                        pltpu.make_async_copy(nbuf, sh_out.at[pidx],
                                              psem).wait()

                    @pl.when(j + 1 < NCHUNK)
                    def _():
                        nidx = src_f.at[pl.ds((j + 1) * CH, CH)]
                        pltpu.async_copy(h_hbm.at[nidx], nbuf, nsem)
                        pltpu.async_copy(sh_s.at[nidx], nsv, nscsem)
                        pltpu.async_copy(sh_d.at[dst2.at[j + 1]], ndv,
                                         nscsem)

                    @pl.when(j >= 2)
                    def _():
                        pltpu.make_async_copy(wb, sh_den.at[dst2.at[j - 2]],
                                              dnsem).wait()

                    pltpu.make_async_copy(sh_s.at[sidx], sv, scsem).wait()
                    pltpu.make_async_copy(sh_d.at[didx], dv, scsem).wait()
                    for k in range(CH // 16):
                        sl = pl.ds(k * 16, 16)
                        e = sv[sl] + dv[sl]
                        e = jnp.where(e > 0, e, e * 0.2)
                        wb[sl] = jnp.exp(e)
                    pltpu.async_copy(wb, sh_den.at[didx], dnsem, add=True)

                    pltpu.make_async_copy(h_hbm.at[sidx], buf, sem).wait()

                    def _scale(i4, carry2):
                        for r in range(4):
                            i = i4 * 4 + r
                            wv = plsc.load_gather(
                                wb, [jnp.full((16,), i, jnp.int32)])
                            for k in range(HALF // 16):
                                sl = pl.ds(k * 16, 16)
                                buf[i, sl] = buf[i, sl] * wv
                        return carry2

                    lax.fori_loop(0, CH // 4, _scale, 0)

                    pltpu.async_copy(buf, sh_out.at[didx], ssem, add=True)
            return carry

        lax.fori_loop(0, (NCHUNK + 1) // 2, _outer, 0)
        pltpu.make_async_copy(rows_a, sh_out.at[dst2.at[NCHUNK - 1]],
                              ssem_a).wait()
        pltpu.make_async_copy(w_b, sh_den.at[dst2.at[NCHUNK - 2]],
                              dnsem_b).wait()
        pltpu.make_async_copy(w_a, sh_den.at[dst2.at[NCHUNK - 1]],
                              dnsem_a).wait()

    @pl.when(c == 0)
    def _():
        _process(h0)

    @pl.when(c == 1)
    def _():
        _process(h1)

    plsc.subcore_barrier()

    @pl.when(c == 0)
    def _():
        pltpu.sync_copy(sh_den.at[pl.ds(node_base, NPT)],
                        den_hbm.at[pl.ds(node_base, NPT)])

    def _drain(o_hbm):
        @pl.when(t < NS - 1)
        def _():
            pltpu.sync_copy(sh_out.at[pl.ds(node_base, NPT)],
                            o_hbm.at[pl.ds(node_base, NPT)])

        @pl.when(t == NS - 1)
        def _():
            pltpu.sync_copy(sh_out.at[pl.ds((NS - 1) * NPT, LAST_NPT)],
                            o_hbm.at[pl.ds((NS - 1) * NPT, LAST_NPT)])

    @pl.when(c == 0)
    def _():
        _drain(o0)

    @pl.when(c == 1)
    def _():
        _drain(o1)


def _sc_gat(h0, h1, s, d, src, dst2):
    mesh = plsc.VectorSubcoreMesh(core_axis_name="c", subcore_axis_name="s",
                                  num_cores=NC, num_subcores=NS)
    f = pl.kernel(
        _sc_body,
        out_type=[
            jax.ShapeDtypeStruct((N, HALF), jnp.float32),
            jax.ShapeDtypeStruct((N, HALF), jnp.float32),
            jax.ShapeDtypeStruct((NPAD,), jnp.float32),
        ],
        mesh=mesh,
        compiler_params=pltpu.CompilerParams(needs_layout_passes=False),
        scratch_types=[
            pltpu.VMEM((EPT,), jnp.int32),
            pltpu.VMEM((NCHUNK, CH), jnp.int32),
            pltpu.VMEM((CH,), jnp.float32),
            pltpu.VMEM((CH,), jnp.float32),
            pltpu.VMEM((CH,), jnp.float32),
            pltpu.VMEM((CH,), jnp.float32),
            pltpu.VMEM((CH,), jnp.float32),
            pltpu.VMEM((CH,), jnp.float32),
            pltpu.VMEM((CH, HALF), jnp.float32),
            pltpu.VMEM((CH, HALF), jnp.float32),
            pltpu.VMEM((NPT,), jnp.float32),
            pltpu.VMEM_SHARED((NPAD,), jnp.float32),
            pltpu.VMEM_SHARED((NPAD,), jnp.float32),
            pltpu.VMEM_SHARED((NPAD,), jnp.float32),
            pltpu.VMEM_SHARED((N, HALF), jnp.float32),
            pltpu.SemaphoreType.DMA,
            pltpu.SemaphoreType.DMA,
            pltpu.SemaphoreType.DMA,
            pltpu.SemaphoreType.DMA,
            pltpu.SemaphoreType.DMA,
            pltpu.SemaphoreType.DMA,
            pltpu.SemaphoreType.DMA,
            pltpu.SemaphoreType.DMA,
        ],
    )
    return f(h0, h1, s, d, src, dst2)


def _tc2_body(o0_ref, o1_ref, den_ref, bg_ref, w1_ref, b1_ref, w2_ref,
              b2_ref, y_ref):
    recip = 1.0 / jnp.maximum(den_ref[...], 1e-30)
    g = jnp.concatenate([o0_ref[...], o1_ref[...]], axis=1)
    g = g * recip + bg_ref[...]
    g = jnp.where(g > 0, g, jnp.exp(g) - 1.0)
    z = jnp.dot(g, w1_ref[...], preferred_element_type=jnp.float32)
    z = jnp.maximum(z + b1_ref[...], 0.0)
    y = jnp.dot(z, w2_ref[...], preferred_element_type=jnp.float32)
    y_ref[...] = y[:, :8] + b2_ref[...]


def _tc2(o0, o1, den, bg, w1p, b1p, w2p, b2p):
    return pl.pallas_call(
        _tc2_body,
        grid=(GRID,),
        in_specs=[
            pl.BlockSpec((RB, HALF), lambda i: (i, 0)),
            pl.BlockSpec((RB, HALF), lambda i: (i, 0)),
            pl.BlockSpec((RB, 1), lambda i: (i, 0)),
            pl.BlockSpec((1, OUT), lambda i: (0, 0)),
            pl.BlockSpec((OUT, HALF), lambda i: (0, 0)),
            pl.BlockSpec((1, HALF), lambda i: (0, 0)),
            pl.BlockSpec((HALF, HALF), lambda i: (0, 0)),
            pl.BlockSpec((1, 8), lambda i: (0, 0)),
        ],
        out_specs=pl.BlockSpec((RB, 8), lambda i: (i, 0)),
        out_shape=jax.ShapeDtypeStruct((N, 8), jnp.float32),
    )(o0, o1, den, bg, w1p, b1p, w2p, b2p)


@jax.jit
def kernel(x, edge_index, W_gat, a_src, a_dst, b_gat, W1, b1, W2, b2):
    src = edge_index[0]
    dst = edge_index[1]

    wext = jnp.concatenate(
        [W_gat, (W_gat @ a_src)[:, None], (W_gat @ a_dst)[:, None],
         jnp.zeros((IN, HALF - 2), jnp.float32)], axis=1)
    h1cols = W1.shape[1]
    w1p = jnp.pad(W1, ((0, 0), (0, HALF - h1cols)))
    b1p = jnp.pad(b1, (0, HALF - h1cols))[None, :]
    w2p = jnp.pad(W2, ((0, HALF - h1cols), (0, HALF - W2.shape[1])))
    b2p = jnp.pad(b2, (0, 8 - W2.shape[1]))[None, :]
    bg = b_gat[None, :]

    h0, h1, sd = _tc1(x, wext)
    s = jnp.pad(sd[:, 0], (0, NPAD - N))
    d = jnp.pad(sd[:, 1], (0, NPAD - N))
    dst2 = dst.reshape(NS, NCHUNK, CH)

    o0, o1, den = _sc_gat(h0, h1, s, d, src, dst2)

    y = _tc2(o0, o1, den[:N, None], bg, w1p, b1p, w2p, b2p)
    return y[:, :W2.shape[1]]


# speedup vs baseline: 19.8251x; 1.0008x over previous
import functools

import jax
import jax.numpy as jnp
from jax import lax
from jax.experimental import pallas as pl
from jax.experimental.pallas import tpu as pltpu
from jax.experimental.pallas import tpu_sc as plsc

N = 10000
E = 160000
IN = 256
OUT = 256
HALF = 128

NC = 2
NS = 16
EPT = E // NS
CH = 80
NCHUNK = EPT // CH
NPT = 640
NPAD = NS * NPT
LAST_NPT = 400
ROWCH = 80

RB = 1000
GRID = N // RB


def _tc1_body(x_ref, wext_ref, h0_ref, h1_ref, sd_ref):
    he = jnp.dot(x_ref[...], wext_ref[...], preferred_element_type=jnp.float32)
    h0_ref[...] = he[:, :HALF]
    h1_ref[...] = he[:, HALF:OUT]
    sd_ref[...] = he[:, OUT:]


def _tc1(x, wext):
    return pl.pallas_call(
        _tc1_body,
        grid=(GRID,),
        in_specs=[
            pl.BlockSpec((RB, IN), lambda i: (i, 0)),
            pl.BlockSpec((IN, OUT + HALF), lambda i: (0, 0)),
        ],
        out_specs=[
            pl.BlockSpec((RB, HALF), lambda i: (i, 0)),
            pl.BlockSpec((RB, HALF), lambda i: (i, 0)),
            pl.BlockSpec((RB, HALF), lambda i: (i, 0)),
        ],
        out_shape=[
            jax.ShapeDtypeStruct((N, HALF), jnp.float32),
            jax.ShapeDtypeStruct((N, HALF), jnp.float32),
            jax.ShapeDtypeStruct((N, HALF), jnp.float32),
        ],
    )(x, wext)


def _sc_body(h0, h1, s_hbm, d_hbm, src_hbm, dst2_hbm,
             o0, o1, den_hbm,
             src_f, dst2, sval_a, sval_b, dval_a, dval_b, w_a, w_b,
             rows_a, rows_b, zvec,
             sh_s, sh_d, sh_den, sh_out,
             sem_a, sem_b, ssem_a, ssem_b, scsem_a, scsem_b,
             dnsem_a, dnsem_b):
    c = lax.axis_index("c")
    t = lax.axis_index("s")

    pltpu.sync_copy(src_hbm.at[pl.ds(t * EPT, EPT)], src_f)
    pltpu.sync_copy(dst2_hbm.at[t], dst2)

    node_base = t * NPT
    nq = jnp.where(t == NS - 1, LAST_NPT // ROWCH, NPT // ROWCH)

    pltpu.sync_copy(s_hbm.at[pl.ds(node_base, NPT)],
                    sh_s.at[pl.ds(node_base, NPT)])
    pltpu.sync_copy(d_hbm.at[pl.ds(node_base, NPT)],
                    sh_d.at[pl.ds(node_base, NPT)])

    zeros16 = jnp.zeros((16,), jnp.float32)

    def _zero_zvec(i, carry):
        zvec[pl.ds(i * 16, 16)] = zeros16
        return carry

    lax.fori_loop(0, NPT // 16, _zero_zvec, 0)

    def _zero_rows(i, carry):
        for k in range(HALF // 16):
            rows_a[i, pl.ds(k * 16, 16)] = zeros16
        return carry

    lax.fori_loop(0, CH, _zero_rows, 0)

    pltpu.sync_copy(zvec, sh_den.at[pl.ds(node_base, NPT)])

    def _zero_out(q, carry):
        pltpu.sync_copy(rows_a,
                        sh_out.at[pl.ds(node_base + q * ROWCH, ROWCH)])
        return carry

    lax.fori_loop(0, nq, _zero_out, 0)

    plsc.subcore_barrier()

    def _process(h_hbm):
        pltpu.async_copy(h_hbm.at[src_f.at[pl.ds(0, CH)]], rows_a, sem_a)
        pltpu.async_copy(sh_s.at[src_f.at[pl.ds(0, CH)]], sval_a, scsem_a)
        pltpu.async_copy(sh_d.at[dst2.at[0]], dval_a, scsem_a)

        def _outer(p, carry):
            for b in range(2):
                j = p * 2 + b
                buf, sem, ssem = ((rows_a, sem_a, ssem_a) if b == 0
                                  else (rows_b, sem_b, ssem_b))
                sv, dv, wb, scsem, dnsem = (
                    (sval_a, dval_a, w_a, scsem_a, dnsem_a) if b == 0
                    else (sval_b, dval_b, w_b, scsem_b, dnsem_b))
                nbuf, nsem = (rows_b, sem_b) if b == 0 else (rows_a, sem_a)
                nsv, ndv, nscsem = ((sval_b, dval_b, scsem_b) if b == 0
                                    else (sval_a, dval_a, scsem_a))
                psem = ssem_b if b == 0 else ssem_a

                @pl.when(j < NCHUNK)
                def _():
                    sidx = src_f.at[pl.ds(j * CH, CH)]
                    didx = dst2.at[j]

                    @pl.when(j >= 1)
                    def _():
                        pidx = dst2.at[j - 1]
                        pltpu.make_async_copy(nbuf, sh_out.at[pidx],
                                              psem).wait()

                    @pl.when(j + 1 < NCHUNK)
                    def _():
                        nidx = src_f.at[pl.ds((j + 1) * CH, CH)]
                        pltpu.async_copy(h_hbm.at[nidx], nbuf, nsem)
                        pltpu.async_copy(sh_s.at[nidx], nsv, nscsem)
                        pltpu.async_copy(sh_d.at[dst2.at[j + 1]], ndv,
                                         nscsem)

                    @pl.when(j >= 2)
                    def _():
                        pltpu.make_async_copy(wb, sh_den.at[dst2.at[j - 2]],
                                              dnsem).wait()

                    pltpu.make_async_copy(sh_s.at[sidx], sv, scsem).wait()
                    pltpu.make_async_copy(sh_d.at[didx], dv, scsem).wait()
                    for k in range(CH // 16):
                        sl = pl.ds(k * 16, 16)
                        e = sv[sl] + dv[sl]
                        e = jnp.where(e > 0, e, e * 0.2)
                        wb[sl] = jnp.exp(e)
                    pltpu.async_copy(wb, sh_den.at[didx], dnsem, add=True)

                    pltpu.make_async_copy(h_hbm.at[sidx], buf, sem).wait()

                    def _scale(i8, carry2):
                        for r in range(8):
                            i = i8 * 8 + r
                            wv = plsc.load_gather(
                                wb, [jnp.full((16,), i, jnp.int32)])
                            for k in range(HALF // 16):
                                sl = pl.ds(k * 16, 16)
                                buf[i, sl] = buf[i, sl] * wv
                        return carry2

                    lax.fori_loop(0, CH // 8, _scale, 0)

                    pltpu.async_copy(buf, sh_out.at[didx], ssem, add=True)
            return carry

        lax.fori_loop(0, (NCHUNK + 1) // 2, _outer, 0)
        pltpu.make_async_copy(rows_a, sh_out.at[dst2.at[NCHUNK - 1]],
                              ssem_a).wait()
        pltpu.make_async_copy(w_b, sh_den.at[dst2.at[NCHUNK - 2]],
                              dnsem_b).wait()
        pltpu.make_async_copy(w_a, sh_den.at[dst2.at[NCHUNK - 1]],
                              dnsem_a).wait()

    @pl.when(c == 0)
    def _():
        _process(h0)

    @pl.when(c == 1)
    def _():
        _process(h1)

    plsc.subcore_barrier()

    @pl.when(c == 0)
    def _():
        pltpu.sync_copy(sh_den.at[pl.ds(node_base, NPT)],
                        den_hbm.at[pl.ds(node_base, NPT)])

    def _drain(o_hbm):
        @pl.when(t < NS - 1)
        def _():
            pltpu.sync_copy(sh_out.at[pl.ds(node_base, NPT)],
                            o_hbm.at[pl.ds(node_base, NPT)])

        @pl.when(t == NS - 1)
        def _():
            pltpu.sync_copy(sh_out.at[pl.ds((NS - 1) * NPT, LAST_NPT)],
                            o_hbm.at[pl.ds((NS - 1) * NPT, LAST_NPT)])

    @pl.when(c == 0)
    def _():
        _drain(o0)

    @pl.when(c == 1)
    def _():
        _drain(o1)


def _sc_gat(h0, h1, s, d, src, dst2):
    mesh = plsc.VectorSubcoreMesh(core_axis_name="c", subcore_axis_name="s",
                                  num_cores=NC, num_subcores=NS)
    f = pl.kernel(
        _sc_body,
        out_type=[
            jax.ShapeDtypeStruct((N, HALF), jnp.float32),
            jax.ShapeDtypeStruct((N, HALF), jnp.float32),
            jax.ShapeDtypeStruct((NPAD,), jnp.float32),
        ],
        mesh=mesh,
        compiler_params=pltpu.CompilerParams(needs_layout_passes=False),
        scratch_types=[
            pltpu.VMEM((EPT,), jnp.int32),
            pltpu.VMEM((NCHUNK, CH), jnp.int32),
            pltpu.VMEM((CH,), jnp.float32),
            pltpu.VMEM((CH,), jnp.float32),
            pltpu.VMEM((CH,), jnp.float32),
            pltpu.VMEM((CH,), jnp.float32),
            pltpu.VMEM((CH,), jnp.float32),
            pltpu.VMEM((CH,), jnp.float32),
            pltpu.VMEM((CH, HALF), jnp.float32),
            pltpu.VMEM((CH, HALF), jnp.float32),
            pltpu.VMEM((NPT,), jnp.float32),
            pltpu.VMEM_SHARED((NPAD,), jnp.float32),
            pltpu.VMEM_SHARED((NPAD,), jnp.float32),
            pltpu.VMEM_SHARED((NPAD,), jnp.float32),
            pltpu.VMEM_SHARED((N, HALF), jnp.float32),
            pltpu.SemaphoreType.DMA,
            pltpu.SemaphoreType.DMA,
            pltpu.SemaphoreType.DMA,
            pltpu.SemaphoreType.DMA,
            pltpu.SemaphoreType.DMA,
            pltpu.SemaphoreType.DMA,
            pltpu.SemaphoreType.DMA,
            pltpu.SemaphoreType.DMA,
        ],
    )
    return f(h0, h1, s, d, src, dst2)


def _tc2_body(o0_ref, o1_ref, den_ref, bg_ref, w1_ref, b1_ref, w2_ref,
              b2_ref, y_ref):
    recip = 1.0 / jnp.maximum(den_ref[...], 1e-30)
    g = jnp.concatenate([o0_ref[...], o1_ref[...]], axis=1)
    g = g * recip + bg_ref[...]
    g = jnp.where(g > 0, g, jnp.exp(g) - 1.0)
    z = jnp.dot(g, w1_ref[...], preferred_element_type=jnp.float32)
    z = jnp.maximum(z + b1_ref[...], 0.0)
    y = jnp.dot(z, w2_ref[...], preferred_element_type=jnp.float32)
    y_ref[...] = y[:, :8] + b2_ref[...]


def _tc2(o0, o1, den, bg, w1p, b1p, w2p, b2p):
    return pl.pallas_call(
        _tc2_body,
        grid=(GRID,),
        in_specs=[
            pl.BlockSpec((RB, HALF), lambda i: (i, 0)),
            pl.BlockSpec((RB, HALF), lambda i: (i, 0)),
            pl.BlockSpec((RB, 1), lambda i: (i, 0)),
            pl.BlockSpec((1, OUT), lambda i: (0, 0)),
            pl.BlockSpec((OUT, HALF), lambda i: (0, 0)),
            pl.BlockSpec((1, HALF), lambda i: (0, 0)),
            pl.BlockSpec((HALF, HALF), lambda i: (0, 0)),
            pl.BlockSpec((1, 8), lambda i: (0, 0)),
        ],
        out_specs=pl.BlockSpec((RB, 8), lambda i: (i, 0)),
        out_shape=jax.ShapeDtypeStruct((N, 8), jnp.float32),
    )(o0, o1, den, bg, w1p, b1p, w2p, b2p)


@jax.jit
def kernel(x, edge_index, W_gat, a_src, a_dst, b_gat, W1, b1, W2, b2):
    src = edge_index[0]
    dst = edge_index[1]

    wext = jnp.concatenate(
        [W_gat, (W_gat @ a_src)[:, None], (W_gat @ a_dst)[:, None],
         jnp.zeros((IN, HALF - 2), jnp.float32)], axis=1)
    h1cols = W1.shape[1]
    w1p = jnp.pad(W1, ((0, 0), (0, HALF - h1cols)))
    b1p = jnp.pad(b1, (0, HALF - h1cols))[None, :]
    w2p = jnp.pad(W2, ((0, HALF - h1cols), (0, HALF - W2.shape[1])))
    b2p = jnp.pad(b2, (0, 8 - W2.shape[1]))[None, :]
    bg = b_gat[None, :]

    h0, h1, sd = _tc1(x, wext)
    s = jnp.pad(sd[:, 0], (0, NPAD - N))
    d = jnp.pad(sd[:, 1], (0, NPAD - N))
    dst2 = dst.reshape(NS, NCHUNK, CH)

    o0, o1, den = _sc_gat(h0, h1, s, d, src, dst2)

    y = _tc2(o0, o1, den[:N, None], bg, w1p, b1p, w2p, b2p)
    return y[:, :W2.shape[1]]


# speedup vs baseline: 19.9077x; 1.0042x over previous
import functools

import jax
import jax.numpy as jnp
from jax import lax
from jax.experimental import pallas as pl
from jax.experimental.pallas import tpu as pltpu
from jax.experimental.pallas import tpu_sc as plsc

N = 10000
E = 160000
IN = 256
OUT = 256
HALF = 128

NC = 2
NS = 16
EPT = E // NS
CH = 80
NCHUNK = EPT // CH
NPT = 640
NPAD = NS * NPT
LAST_NPT = 400
ROWCH = 80

RB = 1000
GRID = N // RB


def _tc1_body(x_ref, wext_ref, h0_ref, h1_ref, sd_ref):
    he = jnp.dot(x_ref[...], wext_ref[...], preferred_element_type=jnp.float32)
    h0_ref[...] = he[:, :HALF]
    h1_ref[...] = he[:, HALF:OUT]
    sd_ref[...] = he[:, OUT:]


def _tc1(x, wext):
    return pl.pallas_call(
        _tc1_body,
        grid=(GRID,),
        in_specs=[
            pl.BlockSpec((RB, IN), lambda i: (i, 0)),
            pl.BlockSpec((IN, OUT + HALF), lambda i: (0, 0)),
        ],
        out_specs=[
            pl.BlockSpec((RB, HALF), lambda i: (i, 0)),
            pl.BlockSpec((RB, HALF), lambda i: (i, 0)),
            pl.BlockSpec((RB, HALF), lambda i: (i, 0)),
        ],
        out_shape=[
            jax.ShapeDtypeStruct((N, HALF), jnp.float32),
            jax.ShapeDtypeStruct((N, HALF), jnp.float32),
            jax.ShapeDtypeStruct((N, HALF), jnp.float32),
        ],
    )(x, wext)


def _sc_body(h0, h1, s_hbm, d_hbm, src_hbm, dst2_hbm,
             o0, o1, den_hbm,
             src_f, dst2, sval_a, sval_b, dval_a, dval_b, w_a, w_b,
             rows_a, rows_b, zvec,
             sh_s, sh_d, sh_den, sh_out,
             sem_a, sem_b, ssem_a, ssem_b, scsem_a, scsem_b,
             dnsem_a, dnsem_b):
    c = lax.axis_index("c")
    t = lax.axis_index("s")

    pltpu.sync_copy(src_hbm.at[pl.ds(t * EPT, EPT)], src_f)
    pltpu.sync_copy(dst2_hbm.at[t], dst2)

    node_base = t * NPT
    nq = jnp.where(t == NS - 1, LAST_NPT // ROWCH, NPT // ROWCH)

    pltpu.sync_copy(s_hbm.at[pl.ds(node_base, NPT)],
                    sh_s.at[pl.ds(node_base, NPT)])
    pltpu.sync_copy(d_hbm.at[pl.ds(node_base, NPT)],
                    sh_d.at[pl.ds(node_base, NPT)])

    zeros16 = jnp.zeros((16,), jnp.float32)

    def _zero_zvec(i, carry):
        zvec[pl.ds(i * 16, 16)] = zeros16
        return carry

    lax.fori_loop(0, NPT // 16, _zero_zvec, 0)

    def _zero_rows(i, carry):
        for k in range(HALF // 16):
            rows_a[i, pl.ds(k * 16, 16)] = zeros16
        return carry

    lax.fori_loop(0, CH, _zero_rows, 0)

    pltpu.sync_copy(zvec, sh_den.at[pl.ds(node_base, NPT)])

    def _zero_out(q, carry):
        pltpu.sync_copy(rows_a,
                        sh_out.at[pl.ds(node_base + q * ROWCH, ROWCH)])
        return carry

    lax.fori_loop(0, nq, _zero_out, 0)

    plsc.subcore_barrier()

    def _process(h_hbm):
        pltpu.async_copy(h_hbm.at[src_f.at[pl.ds(0, CH)]], rows_a, sem_a)
        pltpu.async_copy(sh_s.at[src_f.at[pl.ds(0, CH)]], sval_a, scsem_a)
        pltpu.async_copy(sh_d.at[dst2.at[0]], dval_a, scsem_a)

        def _outer(p, carry):
            for b in range(2):
                j = p * 2 + b
                buf, sem, ssem = ((rows_a, sem_a, ssem_a) if b == 0
                                  else (rows_b, sem_b, ssem_b))
                sv, dv, wb, scsem, dnsem = (
                    (sval_a, dval_a, w_a, scsem_a, dnsem_a) if b == 0
                    else (sval_b, dval_b, w_b, scsem_b, dnsem_b))
                nbuf, nsem = (rows_b, sem_b) if b == 0 else (rows_a, sem_a)
                nsv, ndv, nscsem = ((sval_b, dval_b, scsem_b) if b == 0
                                    else (sval_a, dval_a, scsem_a))
                psem = ssem_b if b == 0 else ssem_a

                @pl.when(j < NCHUNK)
                def _():
                    sidx = src_f.at[pl.ds(j * CH, CH)]
                    didx = dst2.at[j]

                    @pl.when(j + 1 < NCHUNK)
                    def _():
                        nidx = src_f.at[pl.ds((j + 1) * CH, CH)]
                        pltpu.async_copy(sh_s.at[nidx], nsv, nscsem)
                        pltpu.async_copy(sh_d.at[dst2.at[j + 1]], ndv,
                                         nscsem)

                    @pl.when(j >= 2)
                    def _():
                        pltpu.make_async_copy(wb, sh_den.at[dst2.at[j - 2]],
                                              dnsem).wait()

                    pltpu.make_async_copy(sh_s.at[sidx], sv, scsem).wait()
                    pltpu.make_async_copy(sh_d.at[didx], dv, scsem).wait()
                    for k in range(CH // 16):
                        sl = pl.ds(k * 16, 16)
                        e = sv[sl] + dv[sl]
                        e = jnp.where(e > 0, e, e * 0.2)
                        wb[sl] = jnp.exp(e)
                    pltpu.async_copy(wb, sh_den.at[didx], dnsem, add=True)

                    @pl.when(j >= 1)
                    def _():
                        pidx = dst2.at[j - 1]
                        pltpu.make_async_copy(nbuf, sh_out.at[pidx],
                                              psem).wait()

                    @pl.when(j + 1 < NCHUNK)
                    def _():
                        nidx = src_f.at[pl.ds((j + 1) * CH, CH)]
                        pltpu.async_copy(h_hbm.at[nidx], nbuf, nsem)

                    pltpu.make_async_copy(h_hbm.at[sidx], buf, sem).wait()

                    def _scale(i8, carry2):
                        for r in range(8):
                            i = i8 * 8 + r
                            wv = plsc.load_gather(
                                wb, [jnp.full((16,), i, jnp.int32)])
                            for k in range(HALF // 16):
                                sl = pl.ds(k * 16, 16)
                                buf[i, sl] = buf[i, sl] * wv
                        return carry2

                    lax.fori_loop(0, CH // 8, _scale, 0)

                    pltpu.async_copy(buf, sh_out.at[didx], ssem, add=True)
            return carry

        lax.fori_loop(0, (NCHUNK + 1) // 2, _outer, 0)
        pltpu.make_async_copy(rows_a, sh_out.at[dst2.at[NCHUNK - 1]],
                              ssem_a).wait()
        pltpu.make_async_copy(w_b, sh_den.at[dst2.at[NCHUNK - 2]],
                              dnsem_b).wait()
        pltpu.make_async_copy(w_a, sh_den.at[dst2.at[NCHUNK - 1]],
                              dnsem_a).wait()

    @pl.when(c == 0)
    def _():
        _process(h0)

    @pl.when(c == 1)
    def _():
        _process(h1)

    plsc.subcore_barrier()

    @pl.when(c == 0)
    def _():
        pltpu.sync_copy(sh_den.at[pl.ds(node_base, NPT)],
                        den_hbm.at[pl.ds(node_base, NPT)])

    def _drain(o_hbm):
        @pl.when(t < NS - 1)
        def _():
            pltpu.sync_copy(sh_out.at[pl.ds(node_base, NPT)],
                            o_hbm.at[pl.ds(node_base, NPT)])

        @pl.when(t == NS - 1)
        def _():
            pltpu.sync_copy(sh_out.at[pl.ds((NS - 1) * NPT, LAST_NPT)],
                            o_hbm.at[pl.ds((NS - 1) * NPT, LAST_NPT)])

    @pl.when(c == 0)
    def _():
        _drain(o0)

    @pl.when(c == 1)
    def _():
        _drain(o1)


def _sc_gat(h0, h1, s, d, src, dst2):
    mesh = plsc.VectorSubcoreMesh(core_axis_name="c", subcore_axis_name="s",
                                  num_cores=NC, num_subcores=NS)
    f = pl.kernel(
        _sc_body,
        out_type=[
            jax.ShapeDtypeStruct((N, HALF), jnp.float32),
            jax.ShapeDtypeStruct((N, HALF), jnp.float32),
            jax.ShapeDtypeStruct((NPAD,), jnp.float32),
        ],
        mesh=mesh,
        compiler_params=pltpu.CompilerParams(needs_layout_passes=False),
        scratch_types=[
            pltpu.VMEM((EPT,), jnp.int32),
            pltpu.VMEM((NCHUNK, CH), jnp.int32),
            pltpu.VMEM((CH,), jnp.float32),
            pltpu.VMEM((CH,), jnp.float32),
            pltpu.VMEM((CH,), jnp.float32),
            pltpu.VMEM((CH,), jnp.float32),
            pltpu.VMEM((CH,), jnp.float32),
            pltpu.VMEM((CH,), jnp.float32),
            pltpu.VMEM((CH, HALF), jnp.float32),
            pltpu.VMEM((CH, HALF), jnp.float32),
            pltpu.VMEM((NPT,), jnp.float32),
            pltpu.VMEM_SHARED((NPAD,), jnp.float32),
            pltpu.VMEM_SHARED((NPAD,), jnp.float32),
            pltpu.VMEM_SHARED((NPAD,), jnp.float32),
            pltpu.VMEM_SHARED((N, HALF), jnp.float32),
            pltpu.SemaphoreType.DMA,
            pltpu.SemaphoreType.DMA,
            pltpu.SemaphoreType.DMA,
            pltpu.SemaphoreType.DMA,
            pltpu.SemaphoreType.DMA,
            pltpu.SemaphoreType.DMA,
            pltpu.SemaphoreType.DMA,
            pltpu.SemaphoreType.DMA,
        ],
    )
    return f(h0, h1, s, d, src, dst2)


def _tc2_body(o0_ref, o1_ref, den_ref, bg_ref, w1_ref, b1_ref, w2_ref,
              b2_ref, y_ref):
    recip = 1.0 / jnp.maximum(den_ref[...], 1e-30)
    g = jnp.concatenate([o0_ref[...], o1_ref[...]], axis=1)
    g = g * recip + bg_ref[...]
    g = jnp.where(g > 0, g, jnp.exp(g) - 1.0)
    z = jnp.dot(g, w1_ref[...], preferred_element_type=jnp.float32)
    z = jnp.maximum(z + b1_ref[...], 0.0)
    y = jnp.dot(z, w2_ref[...], preferred_element_type=jnp.float32)
    y_ref[...] = y[:, :8] + b2_ref[...]


def _tc2(o0, o1, den, bg, w1p, b1p, w2p, b2p):
    return pl.pallas_call(
        _tc2_body,
        grid=(GRID,),
        in_specs=[
            pl.BlockSpec((RB, HALF), lambda i: (i, 0)),
            pl.BlockSpec((RB, HALF), lambda i: (i, 0)),
            pl.BlockSpec((RB, 1), lambda i: (i, 0)),
            pl.BlockSpec((1, OUT), lambda i: (0, 0)),
            pl.BlockSpec((OUT, HALF), lambda i: (0, 0)),
            pl.BlockSpec((1, HALF), lambda i: (0, 0)),
            pl.BlockSpec((HALF, HALF), lambda i: (0, 0)),
            pl.BlockSpec((1, 8), lambda i: (0, 0)),
        ],
        out_specs=pl.BlockSpec((RB, 8), lambda i: (i, 0)),
        out_shape=jax.ShapeDtypeStruct((N, 8), jnp.float32),
    )(o0, o1, den, bg, w1p, b1p, w2p, b2p)


@jax.jit
def kernel(x, edge_index, W_gat, a_src, a_dst, b_gat, W1, b1, W2, b2):
    src = edge_index[0]
    dst = edge_index[1]

    wext = jnp.concatenate(
        [W_gat, (W_gat @ a_src)[:, None], (W_gat @ a_dst)[:, None],
         jnp.zeros((IN, HALF - 2), jnp.float32)], axis=1)
    h1cols = W1.shape[1]
    w1p = jnp.pad(W1, ((0, 0), (0, HALF - h1cols)))
    b1p = jnp.pad(b1, (0, HALF - h1cols))[None, :]
    w2p = jnp.pad(W2, ((0, HALF - h1cols), (0, HALF - W2.shape[1])))
    b2p = jnp.pad(b2, (0, 8 - W2.shape[1]))[None, :]
    bg = b_gat[None, :]

    h0, h1, sd = _tc1(x, wext)
    s = jnp.pad(sd[:, 0], (0, NPAD - N))
    d = jnp.pad(sd[:, 1], (0, NPAD - N))
    dst2 = dst.reshape(NS, NCHUNK, CH)

    o0, o1, den = _sc_gat(h0, h1, s, d, src, dst2)

    y = _tc2(o0, o1, den[:N, None], bg, w1p, b1p, w2p, b2p)
    return y[:, :W2.shape[1]]
